# Initial kernel scaffold; baseline (speedup 1.0000x reference)
import jax
import jax.numpy as jnp
from jax.experimental import pallas as pl


def kernel(out0, out1, out2, targets):
    raise NotImplementedError("write your pallas kernel here")


# speedup vs baseline: 1.5605x; 1.5605x over previous
import functools

import numpy as np
import jax
import jax.numpy as jnp
from jax import lax
from jax.experimental import pallas as pl
from jax.experimental.pallas import tpu as pltpu
from jax.experimental.pallas import tpu_sc as plsc

_IMG = 608.0
_GRIDS = (76, 38, 19)
_ANCH = np.array(
    [[10, 13], [16, 30], [33, 23], [30, 61], [62, 45], [59, 119],
     [116, 90], [156, 198], [373, 326]], dtype=np.float32).reshape(3, 3, 2)
_SCALED = [_ANCH[i] / (_IMG / g) for i, g in enumerate(_GRIDS)]
_B = 32
_NLANE = 16
_NCORE = 2


def _sc_body(tt_hbm, o0_hbm, o1_hbm, o2_hbm, outg_hbm, asn_hbm,
             tv, fv, idxv, gv, av, sem):
    wid = lax.axis_index("s") * _NCORE + lax.axis_index("c")
    chunk = wid // _NLANE
    j = wid % _NLANE
    b = wid
    c16 = chunk * _NLANE

    pltpu.sync_copy(tt_hbm, tv)

    lane = lax.iota(jnp.int32, 16)

    x1 = tv[pl.ds(0 * _B + c16, 16)] / _IMG
    y1 = tv[pl.ds(1 * _B + c16, 16)] / _IMG
    x2 = tv[pl.ds(2 * _B + c16, 16)] / _IMG
    y2 = tv[pl.ds(3 * _B + c16, 16)] / _IMG

    biou, fidx, fx_s, fy_s, rw_s, rh_s = [], [], [], [], [], []
    for i, nG in enumerate(_GRIDS):
        g = jnp.float32(float(nG))
        tx1 = x1 * g
        ty1 = y1 * g
        tx2 = x2 * g
        ty2 = y2 * g
        gx = (tx1 + tx2) / 2.0
        gy = (ty1 + ty2) / 2.0
        gw = tx2 - tx1
        gh = ty2 - ty1
        wh_area = gw * gh

        best_i = None
        best_a = jnp.zeros((16,), jnp.int32)
        for a in range(3):
            w1 = np.float32(_SCALED[i][a, 0])
            h1 = np.float32(_SCALED[i][a, 1])
            ua = np.float32(w1 * h1 + np.float32(1e-16))
            inter = (jnp.minimum(jnp.float32(w1), gw) *
                     jnp.minimum(jnp.float32(h1), gh))
            iou = inter / (jnp.float32(ua) + wh_area - inter)
            if a == 0:
                best_i = iou
            else:
                upd = iou > best_i
                best_a = jnp.where(upd, jnp.int32(a), best_a)
                best_i = jnp.maximum(best_i, iou)
        biou.append(best_i)

        gi = gx.astype(jnp.int32)
        gj = gy.astype(jnp.int32)
        fx_s.append(gx - gi.astype(jnp.float32))
        fy_s.append(gy - gj.astype(jnp.float32))

        w0 = float(_SCALED[i][0, 0]); h0 = float(_SCALED[i][0, 1])
        w1f = float(_SCALED[i][1, 0]); h1f = float(_SCALED[i][1, 1])
        w2f = float(_SCALED[i][2, 0]); h2f = float(_SCALED[i][2, 1])
        aw = jnp.where(best_a == 0, jnp.float32(w0),
                       jnp.where(best_a == 1, jnp.float32(w1f),
                                 jnp.float32(w2f)))
        ah = jnp.where(best_a == 0, jnp.float32(h0),
                       jnp.where(best_a == 1, jnp.float32(h1f),
                                 jnp.float32(h2f)))
        rw_s.append(gw / aw)
        rh_s.append(gh / ah)

        nG2 = nG * nG
        fidx.append((best_a * jnp.int32(5 * nG2)) + gj * jnp.int32(nG)
                    + gi)

    ssel = jnp.zeros((16,), jnp.int32)
    sbest = biou[0]
    for i in (1, 2):
        upd = biou[i] > sbest
        ssel = jnp.where(upd, jnp.int32(i), ssel)
        sbest = jnp.maximum(sbest, biou[i])

    def sel3(vs):
        return jnp.where(ssel == 0, vs[0],
                         jnp.where(ssel == 1, vs[1], vs[2]))

    txc = sel3(fx_s)
    tyc = sel3(fy_s)
    rwc = sel3(rw_s)
    rhc = sel3(rh_s)
    sf = ssel.astype(jnp.float32)

    @pl.when(j == 0)
    def _():
        av[0, :] = txc
        av[1, :] = tyc
        av[2, :] = rwc
        av[3, :] = rhc
        av[4, :] = sf
        z16 = jnp.zeros((16,), jnp.float32)
        av[5, :] = z16
        av[6, :] = z16
        av[7, :] = z16
        pltpu.sync_copy(av, asn_hbm.at[pl.ds(chunk * 8, 8)])

    cpos = jnp.minimum(lane, jnp.int32(4))
    jbc = jnp.full((16,), j, jnp.int32)
    srcs = (o0_hbm, o1_hbm, o2_hbm)
    for i, nG in enumerate(_GRIDS):
        nG2 = nG * nG
        fv[...] = fidx[i]
        bvec = plsc.load_gather(fv, [jbc])
        idxv[...] = bvec + cpos * jnp.int32(nG2) + b * jnp.int32(15 * nG2)
        pltpu.async_copy(srcs[i].at[idxv], gv, sem).wait()
        pltpu.sync_copy(gv, outg_hbm.at[pl.ds((b * 3 + i) * 16, 16)])


_sc_assign = functools.partial(
    pl.kernel,
    out_type=[jax.ShapeDtypeStruct((_B * 3 * 16,), jnp.float32),
              jax.ShapeDtypeStruct((2 * 8, 16), jnp.float32)],
    mesh=plsc.VectorSubcoreMesh(core_axis_name="c", subcore_axis_name="s"),
    scratch_types=[pltpu.VMEM((4 * _B,), jnp.float32),
                   pltpu.VMEM((16,), jnp.int32),
                   pltpu.VMEM((16,), jnp.int32),
                   pltpu.VMEM((16,), jnp.float32),
                   pltpu.VMEM((8, 16), jnp.float32),
                   pltpu.SemaphoreType.DMA],
    compiler_params=pltpu.CompilerParams(needs_layout_passes=False),
)(_sc_body)


def _tc_body(c00, c01, c02, c10, c11, c12, c20, c21, c22, gref, aref,
             o_loss, o_conf, o_off):
    confs = (c00, c01, c02, c10, c11, c12, c20, c21, c22)

    m = None
    for r in confs:
        mx = jnp.max(r[:, 0, 0, :], axis=1, keepdims=True)
        m = mx if m is None else jnp.maximum(m, mx)
    se = None
    for r in confs:
        s = jnp.sum(jnp.exp(r[:, 0, 0, :] - m), axis=1, keepdims=True)
        se = s if se is None else se + s
    lse = jnp.log(se) + m

    g = gref[...]
    a = aref[...]
    sf = a[:, 4:5]

    def pick(c):
        acc = jnp.where(sf == 0.0, g[:, c:c + 1], 0.0)
        acc = acc + jnp.where(sf == 1.0, g[:, 16 + c:16 + c + 1], 0.0)
        acc = acc + jnp.where(sf == 2.0, g[:, 32 + c:32 + c + 1], 0.0)
        return acc

    p0, p1, p2, p3, pc = pick(0), pick(1), pick(2), pick(3), pick(4)
    tx = a[:, 0:1]
    ty = a[:, 1:2]
    tw = jnp.log(a[:, 2:3] + 1e-16)
    th = jnp.log(a[:, 3:4] + 1e-16)

    def sig(x):
        return jnp.clip(jax.nn.sigmoid(x), 0.0001, 1.0 - 0.0001)

    off_per = ((sig(p0) - tx) ** 2 + (sig(p1) - ty) ** 2 +
               (p2 - tw) ** 2 + (p3 - th) ** 2)
    off = jnp.sum(off_per, axis=0, keepdims=True) / jnp.float32(_B)
    lc = jnp.sum(lse - pc, axis=0, keepdims=True) / jnp.float32(_B)
    o_off[...] = off
    o_conf[...] = lc
    o_loss[...] = off + lc


def kernel(out0, out1, out2, targets):
    ttf = targets.T.reshape(-1)
    o0f = out0.reshape(-1)
    o1f = out1.reshape(-1)
    o2f = out2.reshape(-1)
    outg, asn = _sc_assign(ttf, o0f, o1f, o2f)
    outg2 = outg.reshape(_B, 48)
    asn = asn.reshape(2, 8, 16).transpose(0, 2, 1).reshape(_B, 8)[:, :5]

    o0r = out0.reshape(_B, 15, 1, _GRIDS[0] * _GRIDS[0])
    o1r = out1.reshape(_B, 15, 1, _GRIDS[1] * _GRIDS[1])
    o2r = out2.reshape(_B, 15, 1, _GRIDS[2] * _GRIDS[2])

    def conf_spec(n2, ch):
        return pl.BlockSpec((_B, 1, 1, n2), lambda i, c=ch: (0, c, 0, 0))

    n0, n1, n2 = (g * g for g in _GRIDS)
    in_specs = ([conf_spec(n0, c) for c in (4, 9, 14)] +
                [conf_spec(n1, c) for c in (4, 9, 14)] +
                [conf_spec(n2, c) for c in (4, 9, 14)] +
                [pl.BlockSpec((_B, 48), lambda i: (0, 0)),
                 pl.BlockSpec((_B, 5), lambda i: (0, 0))])

    loss, lc, off = pl.pallas_call(
        _tc_body,
        grid=(1,),
        in_specs=in_specs,
        out_specs=[pl.BlockSpec((1, 1), lambda i: (0, 0))] * 3,
        out_shape=[jax.ShapeDtypeStruct((1, 1), jnp.float32)] * 3,
    )(o0r, o0r, o0r, o1r, o1r, o1r, o2r, o2r, o2r, outg2, asn)

    return (loss.reshape(1), lc.reshape(1), off.reshape(1))


# speedup vs baseline: 4.0100x; 2.5697x over previous
import functools

import numpy as np
import jax
import jax.numpy as jnp
from jax import lax
from jax.experimental import pallas as pl
from jax.experimental.pallas import tpu as pltpu
from jax.experimental.pallas import tpu_sc as plsc

_IMG = 608.0
_GRIDS = (76, 38, 19)
_ANCH = np.array(
    [[10, 13], [16, 30], [33, 23], [30, 61], [62, 45], [59, 119],
     [116, 90], [156, 198], [373, 326]], dtype=np.float32).reshape(3, 3, 2)
_SCALED = [_ANCH[i] / (_IMG / g) for i, g in enumerate(_GRIDS)]
_B = 32
_NLANE = 16
_NCORE = 2


def _sc_body(tt_hbm, asn_hbm, tv, av):
    wid = lax.axis_index("s") * _NCORE + lax.axis_index("c")
    chunk = wid // _NLANE
    j = wid % _NLANE
    c16 = chunk * _NLANE

    pltpu.sync_copy(tt_hbm, tv)

    x1 = tv[pl.ds(0 * _B + c16, 16)] / _IMG
    y1 = tv[pl.ds(1 * _B + c16, 16)] / _IMG
    x2 = tv[pl.ds(2 * _B + c16, 16)] / _IMG
    y2 = tv[pl.ds(3 * _B + c16, 16)] / _IMG

    biou, ba_s, gj_s, gi_s, fx_s, fy_s, rw_s, rh_s = ([] for _ in range(8))
    for i, nG in enumerate(_GRIDS):
        g = jnp.float32(float(nG))
        tx1 = x1 * g
        ty1 = y1 * g
        tx2 = x2 * g
        ty2 = y2 * g
        gx = (tx1 + tx2) / 2.0
        gy = (ty1 + ty2) / 2.0
        gw = tx2 - tx1
        gh = ty2 - ty1
        wh_area = gw * gh

        best_i = None
        best_a = jnp.zeros((16,), jnp.int32)
        for a in range(3):
            w1 = np.float32(_SCALED[i][a, 0])
            h1 = np.float32(_SCALED[i][a, 1])
            ua = np.float32(w1 * h1 + np.float32(1e-16))
            inter = (jnp.minimum(jnp.float32(w1), gw) *
                     jnp.minimum(jnp.float32(h1), gh))
            iou = inter / (jnp.float32(ua) + wh_area - inter)
            if a == 0:
                best_i = iou
            else:
                upd = iou > best_i
                best_a = jnp.where(upd, jnp.int32(a), best_a)
                best_i = jnp.maximum(best_i, iou)
        biou.append(best_i)
        ba_s.append(best_a)

        gi = gx.astype(jnp.int32)
        gj = gy.astype(jnp.int32)
        gi_s.append(gi.astype(jnp.float32))
        gj_s.append(gj.astype(jnp.float32))
        fx_s.append(gx - gi.astype(jnp.float32))
        fy_s.append(gy - gj.astype(jnp.float32))

        w0 = float(_SCALED[i][0, 0]); h0 = float(_SCALED[i][0, 1])
        w1f = float(_SCALED[i][1, 0]); h1f = float(_SCALED[i][1, 1])
        w2f = float(_SCALED[i][2, 0]); h2f = float(_SCALED[i][2, 1])
        aw = jnp.where(best_a == 0, jnp.float32(w0),
                       jnp.where(best_a == 1, jnp.float32(w1f),
                                 jnp.float32(w2f)))
        ah = jnp.where(best_a == 0, jnp.float32(h0),
                       jnp.where(best_a == 1, jnp.float32(h1f),
                                 jnp.float32(h2f)))
        rw_s.append(gw / aw)
        rh_s.append(gh / ah)

    ssel = jnp.zeros((16,), jnp.int32)
    sbest = biou[0]
    for i in (1, 2):
        upd = biou[i] > sbest
        ssel = jnp.where(upd, jnp.int32(i), ssel)
        sbest = jnp.maximum(sbest, biou[i])

    def sel3(vs):
        return jnp.where(ssel == 0, vs[0],
                         jnp.where(ssel == 1, vs[1], vs[2]))

    @pl.when(j == 0)
    def _():
        av[0, :] = sel3(fx_s)
        av[1, :] = sel3(fy_s)
        av[2, :] = sel3(rw_s)
        av[3, :] = sel3(rh_s)
        av[4, :] = ssel.astype(jnp.float32)
        av[5, :] = sel3(ba_s).astype(jnp.float32)
        av[6, :] = sel3(gj_s)
        av[7, :] = sel3(gi_s)
        pltpu.sync_copy(av, asn_hbm.at[pl.ds(chunk * 8, 8)])


_sc_assign = functools.partial(
    pl.kernel,
    out_type=jax.ShapeDtypeStruct((16, 16), jnp.float32),
    mesh=plsc.VectorSubcoreMesh(core_axis_name="c", subcore_axis_name="s"),
    scratch_types=[pltpu.VMEM((4 * _B,), jnp.float32),
                   pltpu.VMEM((8, 16), jnp.float32)],
    compiler_params=pltpu.CompilerParams(needs_layout_passes=False),
)(_sc_body)


def _lse_body(c00, c01, c02, c10, c11, c12, c20, c21, c22, o_lse):
    confs = (c00, c01, c02, c10, c11, c12, c20, c21, c22)
    m = None
    for r in confs:
        x = r[:, 0, :, :]
        m1 = jnp.max(x, axis=2)
        mx = jnp.max(m1, axis=1, keepdims=True)
        m = mx if m is None else jnp.maximum(m, mx)
    se = None
    for r in confs:
        x = r[:, 0, :, :]
        e = jnp.exp(x - m[:, :, None])
        s1 = jnp.sum(e, axis=2)
        s = jnp.sum(s1, axis=1, keepdims=True)
        se = s if se is None else se + s
    o_lse[...] = jnp.log(se) + m


def _mk_copy(r, aref, o0, o1, o2, gs, sem):
    chunk, lane = divmod(r, _NLANE)
    base = chunk * 8
    si = aref[base + 4, lane].astype(jnp.int32)
    a5 = aref[base + 5, lane].astype(jnp.int32) * 5
    gj = aref[base + 6, lane].astype(jnp.int32)
    gj8 = pl.multiple_of((gj // 8) * 8, 8)
    srcs = (o0, o1, o2)
    out = []
    for i in range(3):
        cp = pltpu.make_async_copy(
            srcs[i].at[r, pl.ds(a5, 5), pl.ds(gj8, 8), :],
            gs[i].at[r],
            sem)
        out.append((si == i, cp))
    return out


def _comb_body(aref, lse_ref, o0, o1, o2, o_loss, o_conf, o_off,
               g0, g1, g2, sem):
    gs = (g0, g1, g2)
    for r in range(_B):
        for cond, cp in _mk_copy(r, aref, o0, o1, o2, gs, sem):
            pl.when(cond)(cp.start)
    for r in range(_B):
        for cond, cp in _mk_copy(r, aref, o0, o1, o2, gs, sem):
            pl.when(cond)(cp.wait)

    ri = lax.broadcasted_iota(jnp.int32, (8, 76), 0)
    ci = lax.broadcasted_iota(jnp.int32, (8, 76), 1)
    bi = lax.broadcasted_iota(jnp.int32, (_B, 1), 0)

    p = [jnp.zeros((_B, 1), jnp.float32) for _ in range(5)]
    tx = jnp.zeros((_B, 1), jnp.float32)
    ty = jnp.zeros((_B, 1), jnp.float32)
    rw = jnp.zeros((_B, 1), jnp.float32)
    rh = jnp.zeros((_B, 1), jnp.float32)
    for r in range(_B):
        chunk, lane = divmod(r, _NLANE)
        base = chunk * 8
        sf = aref[base + 4, lane]
        gj = aref[base + 6, lane].astype(jnp.int32)
        gi = aref[base + 7, lane].astype(jnp.int32)
        dj = gj - (gj // 8) * 8
        mask = jnp.where((ri == dj) & (ci == gi), 1.0, 0.0)
        oh = jnp.where(bi == r, 1.0, 0.0)
        for c in range(5):
            vals = [jnp.sum(gs[i][r, c] * mask[:, :nG])
                    for i, nG in enumerate(_GRIDS)]
            val = jnp.where(sf == 0.0, vals[0],
                            jnp.where(sf == 1.0, vals[1], vals[2]))
            p[c] = p[c] + val * oh
        tx = tx + aref[base + 0, lane] * oh
        ty = ty + aref[base + 1, lane] * oh
        rw = rw + aref[base + 2, lane] * oh
        rh = rh + aref[base + 3, lane] * oh

    tw = jnp.log(rw + 1e-16)
    th = jnp.log(rh + 1e-16)

    def sig(x):
        return jnp.clip(jax.nn.sigmoid(x), 0.0001, 1.0 - 0.0001)

    off_per = ((sig(p[0]) - tx) ** 2 + (sig(p[1]) - ty) ** 2 +
               (p[2] - tw) ** 2 + (p[3] - th) ** 2)
    off = jnp.sum(off_per, axis=0, keepdims=True) / jnp.float32(_B)
    lc = jnp.sum(lse_ref[...] - p[4], axis=0, keepdims=True) / jnp.float32(_B)
    o_off[...] = off
    o_conf[...] = lc
    o_loss[...] = off + lc


def kernel(out0, out1, out2, targets):
    ttf = targets.T.reshape(-1)
    asn = _sc_assign(ttf)

    def conf_spec(nG, ch):
        return pl.BlockSpec((_B, 1, nG, nG), lambda i, c=ch: (0, c, 0, 0))

    lse = pl.pallas_call(
        _lse_body,
        grid=(1,),
        in_specs=([conf_spec(_GRIDS[0], c) for c in (4, 9, 14)] +
                  [conf_spec(_GRIDS[1], c) for c in (4, 9, 14)] +
                  [conf_spec(_GRIDS[2], c) for c in (4, 9, 14)]),
        out_specs=pl.BlockSpec((_B, 1), lambda i: (0, 0)),
        out_shape=jax.ShapeDtypeStruct((_B, 1), jnp.float32),
    )(out0, out0, out0, out1, out1, out1, out2, out2, out2)

    loss, lc, off = pl.pallas_call(
        _comb_body,
        grid=(1,),
        in_specs=[pl.BlockSpec(memory_space=pltpu.SMEM),
                  pl.BlockSpec((_B, 1), lambda i: (0, 0)),
                  pl.BlockSpec(memory_space=pl.ANY),
                  pl.BlockSpec(memory_space=pl.ANY),
                  pl.BlockSpec(memory_space=pl.ANY)],
        out_specs=[pl.BlockSpec((1, 1), lambda i: (0, 0))] * 3,
        out_shape=[jax.ShapeDtypeStruct((1, 1), jnp.float32)] * 3,
        scratch_shapes=[pltpu.VMEM((_B, 5, 8, _GRIDS[0]), jnp.float32),
                        pltpu.VMEM((_B, 5, 8, _GRIDS[1]), jnp.float32),
                        pltpu.VMEM((_B, 5, 8, _GRIDS[2]), jnp.float32),
                        pltpu.SemaphoreType.DMA],
    )(asn, lse, out0, out1, out2)

    return (loss.reshape(1), lc.reshape(1), off.reshape(1))


# speedup vs baseline: 4.0704x; 1.0151x over previous
import functools

import numpy as np
import jax
import jax.numpy as jnp
from jax import lax
from jax.experimental import pallas as pl
from jax.experimental.pallas import tpu as pltpu
from jax.experimental.pallas import tpu_sc as plsc

_IMG = 608.0
_GRIDS = (76, 38, 19)
_ANCH = np.array(
    [[10, 13], [16, 30], [33, 23], [30, 61], [62, 45], [59, 119],
     [116, 90], [156, 198], [373, 326]], dtype=np.float32).reshape(3, 3, 2)
_SCALED = [_ANCH[i] / (_IMG / g) for i, g in enumerate(_GRIDS)]
_B = 32
_NLANE = 16
_NCORE = 2


def _sc_body(tt_hbm, asn_hbm, tv, av):
    wid = lax.axis_index("s") * _NCORE + lax.axis_index("c")
    chunk = wid // _NLANE
    j = wid % _NLANE
    c16 = chunk * _NLANE

    pltpu.sync_copy(tt_hbm, tv)

    x1 = tv[pl.ds(0 * _B + c16, 16)] / _IMG
    y1 = tv[pl.ds(1 * _B + c16, 16)] / _IMG
    x2 = tv[pl.ds(2 * _B + c16, 16)] / _IMG
    y2 = tv[pl.ds(3 * _B + c16, 16)] / _IMG

    biou, ba_s, gj_s, gi_s, fx_s, fy_s, rw_s, rh_s = ([] for _ in range(8))
    for i, nG in enumerate(_GRIDS):
        g = jnp.float32(float(nG))
        tx1 = x1 * g
        ty1 = y1 * g
        tx2 = x2 * g
        ty2 = y2 * g
        gx = (tx1 + tx2) / 2.0
        gy = (ty1 + ty2) / 2.0
        gw = tx2 - tx1
        gh = ty2 - ty1
        wh_area = gw * gh

        best_i = None
        best_a = jnp.zeros((16,), jnp.int32)
        for a in range(3):
            w1 = np.float32(_SCALED[i][a, 0])
            h1 = np.float32(_SCALED[i][a, 1])
            ua = np.float32(w1 * h1 + np.float32(1e-16))
            inter = (jnp.minimum(jnp.float32(w1), gw) *
                     jnp.minimum(jnp.float32(h1), gh))
            iou = inter / (jnp.float32(ua) + wh_area - inter)
            if a == 0:
                best_i = iou
            else:
                upd = iou > best_i
                best_a = jnp.where(upd, jnp.int32(a), best_a)
                best_i = jnp.maximum(best_i, iou)
        biou.append(best_i)
        ba_s.append(best_a)

        gi = gx.astype(jnp.int32)
        gj = gy.astype(jnp.int32)
        gi_s.append(gi.astype(jnp.float32))
        gj_s.append(gj.astype(jnp.float32))
        fx_s.append(gx - gi.astype(jnp.float32))
        fy_s.append(gy - gj.astype(jnp.float32))

        w0 = float(_SCALED[i][0, 0]); h0 = float(_SCALED[i][0, 1])
        w1f = float(_SCALED[i][1, 0]); h1f = float(_SCALED[i][1, 1])
        w2f = float(_SCALED[i][2, 0]); h2f = float(_SCALED[i][2, 1])
        aw = jnp.where(best_a == 0, jnp.float32(w0),
                       jnp.where(best_a == 1, jnp.float32(w1f),
                                 jnp.float32(w2f)))
        ah = jnp.where(best_a == 0, jnp.float32(h0),
                       jnp.where(best_a == 1, jnp.float32(h1f),
                                 jnp.float32(h2f)))
        rw_s.append(gw / aw)
        rh_s.append(gh / ah)

    ssel = jnp.zeros((16,), jnp.int32)
    sbest = biou[0]
    for i in (1, 2):
        upd = biou[i] > sbest
        ssel = jnp.where(upd, jnp.int32(i), ssel)
        sbest = jnp.maximum(sbest, biou[i])

    def sel3(vs):
        return jnp.where(ssel == 0, vs[0],
                         jnp.where(ssel == 1, vs[1], vs[2]))

    @pl.when(j == 0)
    def _():
        av[0, :] = sel3(fx_s)
        av[1, :] = sel3(fy_s)
        av[2, :] = sel3(rw_s)
        av[3, :] = sel3(rh_s)
        av[4, :] = ssel.astype(jnp.float32)
        av[5, :] = sel3(ba_s).astype(jnp.float32)
        av[6, :] = sel3(gj_s)
        av[7, :] = sel3(gi_s)
        pltpu.sync_copy(av, asn_hbm.at[pl.ds(chunk * 8, 8)])


_sc_assign = functools.partial(
    pl.kernel,
    out_type=jax.ShapeDtypeStruct((16, 16), jnp.float32),
    mesh=plsc.VectorSubcoreMesh(core_axis_name="c", subcore_axis_name="s"),
    scratch_types=[pltpu.VMEM((4 * _B,), jnp.float32),
                   pltpu.VMEM((8, 16), jnp.float32)],
    compiler_params=pltpu.CompilerParams(needs_layout_passes=False),
)(_sc_body)


def _lse_body(c0, c1, c2, o_lse, m_sc, se_sc):
    first = pl.program_id(0) == 0
    xs = [c0[:, 0, :, :], c1[:, 0, :, :], c2[:, 0, :, :]]
    mx = None
    for x in xs:
        m1 = jnp.max(jnp.max(x, axis=2), axis=1, keepdims=True)
        mx = m1 if mx is None else jnp.maximum(mx, m1)
    m_old = jnp.where(first, jnp.float32(-1e30), m_sc[...])
    se_old = jnp.where(first, jnp.float32(0.0), se_sc[...])
    m_new = jnp.maximum(m_old, mx)
    se = se_old * jnp.exp(m_old - m_new)
    for x in xs:
        e = jnp.exp(x - m_new[:, :, None])
        se = se + jnp.sum(jnp.sum(e, axis=2), axis=1, keepdims=True)
    m_sc[...] = m_new
    se_sc[...] = se

    @pl.when(pl.program_id(0) == 2)
    def _():
        o_lse[...] = jnp.log(se) + m_new


def _mk_copy(r, aref, o0, o1, o2, gs, sem):
    chunk, lane = divmod(r, _NLANE)
    base = chunk * 8
    si = aref[base + 4, lane].astype(jnp.int32)
    a5 = aref[base + 5, lane].astype(jnp.int32) * 5
    gj = aref[base + 6, lane].astype(jnp.int32)
    gj8 = pl.multiple_of((gj // 8) * 8, 8)
    srcs = (o0, o1, o2)
    out = []
    for i in range(3):
        cp = pltpu.make_async_copy(
            srcs[i].at[r, pl.ds(a5, 5), pl.ds(gj8, 8), :],
            gs[i].at[r],
            sem)
        out.append((si == i, cp))
    return out


def _comb_body(aref, lse_ref, o0, o1, o2, o_loss, o_conf, o_off,
               g0, g1, g2, sem):
    gs = (g0, g1, g2)
    for r in range(_B):
        for cond, cp in _mk_copy(r, aref, o0, o1, o2, gs, sem):
            pl.when(cond)(cp.start)
    for r in range(_B):
        for cond, cp in _mk_copy(r, aref, o0, o1, o2, gs, sem):
            pl.when(cond)(cp.wait)

    ri = lax.broadcasted_iota(jnp.int32, (8, 76), 0)
    ci = lax.broadcasted_iota(jnp.int32, (8, 76), 1)
    bi = lax.broadcasted_iota(jnp.int32, (_B, 1), 0)

    p = [jnp.zeros((_B, 1), jnp.float32) for _ in range(5)]
    tx = jnp.zeros((_B, 1), jnp.float32)
    ty = jnp.zeros((_B, 1), jnp.float32)
    rw = jnp.zeros((_B, 1), jnp.float32)
    rh = jnp.zeros((_B, 1), jnp.float32)
    for r in range(_B):
        chunk, lane = divmod(r, _NLANE)
        base = chunk * 8
        sf = aref[base + 4, lane]
        gj = aref[base + 6, lane].astype(jnp.int32)
        gi = aref[base + 7, lane].astype(jnp.int32)
        dj = gj - (gj // 8) * 8
        mask = jnp.where((ri == dj) & (ci == gi), 1.0, 0.0)
        oh = jnp.where(bi == r, 1.0, 0.0)
        for c in range(5):
            vals = [jnp.sum(gs[i][r, c] * mask[:, :nG])
                    for i, nG in enumerate(_GRIDS)]
            val = jnp.where(sf == 0.0, vals[0],
                            jnp.where(sf == 1.0, vals[1], vals[2]))
            p[c] = p[c] + val * oh
        tx = tx + aref[base + 0, lane] * oh
        ty = ty + aref[base + 1, lane] * oh
        rw = rw + aref[base + 2, lane] * oh
        rh = rh + aref[base + 3, lane] * oh

    tw = jnp.log(rw + 1e-16)
    th = jnp.log(rh + 1e-16)

    def sig(x):
        return jnp.clip(jax.nn.sigmoid(x), 0.0001, 1.0 - 0.0001)

    off_per = ((sig(p[0]) - tx) ** 2 + (sig(p[1]) - ty) ** 2 +
               (p[2] - tw) ** 2 + (p[3] - th) ** 2)
    off = jnp.sum(off_per, axis=0, keepdims=True) / jnp.float32(_B)
    lc = jnp.sum(lse_ref[...] - p[4], axis=0, keepdims=True) / jnp.float32(_B)
    o_off[...] = off
    o_conf[...] = lc
    o_loss[...] = off + lc


def kernel(out0, out1, out2, targets):
    ttf = targets.T.reshape(-1)
    asn = _sc_assign(ttf)

    def conf_spec(nG):
        return pl.BlockSpec((_B, 1, nG, nG), lambda i: (0, 4 + 5 * i, 0, 0))

    lse = pl.pallas_call(
        _lse_body,
        grid=(3,),
        in_specs=[conf_spec(g) for g in _GRIDS],
        out_specs=pl.BlockSpec((_B, 1), lambda i: (0, 0)),
        out_shape=jax.ShapeDtypeStruct((_B, 1), jnp.float32),
        scratch_shapes=[pltpu.VMEM((_B, 1), jnp.float32),
                        pltpu.VMEM((_B, 1), jnp.float32)],
    )(out0, out1, out2)

    loss, lc, off = pl.pallas_call(
        _comb_body,
        grid=(1,),
        in_specs=[pl.BlockSpec(memory_space=pltpu.SMEM),
                  pl.BlockSpec((_B, 1), lambda i: (0, 0)),
                  pl.BlockSpec(memory_space=pl.ANY),
                  pl.BlockSpec(memory_space=pl.ANY),
                  pl.BlockSpec(memory_space=pl.ANY)],
        out_specs=[pl.BlockSpec((1, 1), lambda i: (0, 0))] * 3,
        out_shape=[jax.ShapeDtypeStruct((1, 1), jnp.float32)] * 3,
        scratch_shapes=[pltpu.VMEM((_B, 5, 8, _GRIDS[0]), jnp.float32),
                        pltpu.VMEM((_B, 5, 8, _GRIDS[1]), jnp.float32),
                        pltpu.VMEM((_B, 5, 8, _GRIDS[2]), jnp.float32),
                        pltpu.SemaphoreType.DMA],
    )(asn, lse, out0, out1, out2)

    return (loss.reshape(1), lc.reshape(1), off.reshape(1))


# speedup vs baseline: 8.8054x; 2.1632x over previous
import functools

import numpy as np
import jax
import jax.numpy as jnp
from jax import lax
from jax.experimental import pallas as pl
from jax.experimental.pallas import tpu as pltpu
from jax.experimental.pallas import tpu_sc as plsc

_IMG = 608.0
_GRIDS = (76, 38, 19)
_ANCH = np.array(
    [[10, 13], [16, 30], [33, 23], [30, 61], [62, 45], [59, 119],
     [116, 90], [156, 198], [373, 326]], dtype=np.float32).reshape(3, 3, 2)
_SCALED = [_ANCH[i] / (_IMG / g) for i, g in enumerate(_GRIDS)]
_B = 32
_NLANE = 16
_NCORE = 2


def _sc_body(tt_hbm, asn_hbm, tv, av):
    wid = lax.axis_index("s") * _NCORE + lax.axis_index("c")
    chunk = wid // _NLANE
    j = wid % _NLANE
    c16 = chunk * _NLANE

    pltpu.sync_copy(tt_hbm, tv)

    x1 = tv[pl.ds(0 * _B + c16, 16)] / _IMG
    y1 = tv[pl.ds(1 * _B + c16, 16)] / _IMG
    x2 = tv[pl.ds(2 * _B + c16, 16)] / _IMG
    y2 = tv[pl.ds(3 * _B + c16, 16)] / _IMG

    biou, ba_s, gj_s, gi_s, fx_s, fy_s, rw_s, rh_s = ([] for _ in range(8))
    for i, nG in enumerate(_GRIDS):
        g = jnp.float32(float(nG))
        tx1 = x1 * g
        ty1 = y1 * g
        tx2 = x2 * g
        ty2 = y2 * g
        gx = (tx1 + tx2) / 2.0
        gy = (ty1 + ty2) / 2.0
        gw = tx2 - tx1
        gh = ty2 - ty1
        wh_area = gw * gh

        best_i = None
        best_a = jnp.zeros((16,), jnp.int32)
        for a in range(3):
            w1 = np.float32(_SCALED[i][a, 0])
            h1 = np.float32(_SCALED[i][a, 1])
            ua = np.float32(w1 * h1 + np.float32(1e-16))
            inter = (jnp.minimum(jnp.float32(w1), gw) *
                     jnp.minimum(jnp.float32(h1), gh))
            iou = inter / (jnp.float32(ua) + wh_area - inter)
            if a == 0:
                best_i = iou
            else:
                upd = iou > best_i
                best_a = jnp.where(upd, jnp.int32(a), best_a)
                best_i = jnp.maximum(best_i, iou)
        biou.append(best_i)
        ba_s.append(best_a)

        gi = gx.astype(jnp.int32)
        gj = gy.astype(jnp.int32)
        gi_s.append(gi.astype(jnp.float32))
        gj_s.append(gj.astype(jnp.float32))
        fx_s.append(gx - gi.astype(jnp.float32))
        fy_s.append(gy - gj.astype(jnp.float32))

        w0 = float(_SCALED[i][0, 0]); h0 = float(_SCALED[i][0, 1])
        w1f = float(_SCALED[i][1, 0]); h1f = float(_SCALED[i][1, 1])
        w2f = float(_SCALED[i][2, 0]); h2f = float(_SCALED[i][2, 1])
        aw = jnp.where(best_a == 0, jnp.float32(w0),
                       jnp.where(best_a == 1, jnp.float32(w1f),
                                 jnp.float32(w2f)))
        ah = jnp.where(best_a == 0, jnp.float32(h0),
                       jnp.where(best_a == 1, jnp.float32(h1f),
                                 jnp.float32(h2f)))
        rw_s.append(gw / aw)
        rh_s.append(gh / ah)

    ssel = jnp.zeros((16,), jnp.int32)
    sbest = biou[0]
    for i in (1, 2):
        upd = biou[i] > sbest
        ssel = jnp.where(upd, jnp.int32(i), ssel)
        sbest = jnp.maximum(sbest, biou[i])

    def sel3(vs):
        return jnp.where(ssel == 0, vs[0],
                         jnp.where(ssel == 1, vs[1], vs[2]))

    @pl.when(j == 0)
    def _():
        av[0, :] = sel3(fx_s)
        av[1, :] = sel3(fy_s)
        av[2, :] = sel3(rw_s)
        av[3, :] = sel3(rh_s)
        av[4, :] = ssel.astype(jnp.float32)
        av[5, :] = sel3(ba_s).astype(jnp.float32)
        av[6, :] = sel3(gj_s)
        av[7, :] = sel3(gi_s)
        pltpu.sync_copy(av, asn_hbm.at[pl.ds(chunk * 8, 8)])


_sc_assign = functools.partial(
    pl.kernel,
    out_type=jax.ShapeDtypeStruct((16, 16), jnp.float32),
    mesh=plsc.VectorSubcoreMesh(core_axis_name="c", subcore_axis_name="s"),
    scratch_types=[pltpu.VMEM((4 * _B,), jnp.float32),
                   pltpu.VMEM((8, 16), jnp.float32)],
    compiler_params=pltpu.CompilerParams(needs_layout_passes=False),
)(_sc_body)


def _lse_body(c0, c1, c2, o_lse, m_sc, se_sc):
    first = pl.program_id(0) == 0
    x0 = c0[0]
    x1 = c1[0]
    X2 = c2[...]
    x2s = [X2[:, :, 4, :], X2[:, :, 9, :], X2[:, :, 14, :]]

    def rowmax(x, axes):
        return jnp.max(x, axis=axes).reshape(1, _B)

    mx = jnp.maximum(rowmax(x0, (0, 2)), rowmax(x1, (0, 2)))
    mx2 = jnp.maximum(jnp.maximum(rowmax(x2s[0], (0, 1)),
                                  rowmax(x2s[1], (0, 1))),
                      rowmax(x2s[2], (0, 1)))
    mx = jnp.maximum(mx, jnp.where(first, mx2, jnp.float32(-1e30)))

    m_old = jnp.where(first, jnp.float32(-1e30), m_sc[...])
    se_old = jnp.where(first, jnp.float32(0.0), se_sc[...])
    m_new = jnp.maximum(m_old, mx)
    se = se_old * jnp.exp(m_old - m_new)
    mv = m_new.reshape(_B)
    se = se + jnp.sum(jnp.exp(x0 - mv[None, :, None]),
                      axis=(0, 2)).reshape(1, _B)
    se = se + jnp.sum(jnp.exp(x1 - mv[None, :, None]),
                      axis=(0, 2)).reshape(1, _B)
    se2 = None
    for xs in x2s:
        s = jnp.sum(jnp.exp(xs - mv[None, None, :]), axis=(0, 1))
        se2 = s if se2 is None else se2 + s
    se = se + jnp.where(first, se2.reshape(1, _B), jnp.float32(0.0))
    m_sc[...] = m_new
    se_sc[...] = se

    @pl.when(pl.program_id(0) == 2)
    def _():
        o_lse[...] = jnp.log(se) + m_new


def _row_scalars(r, aref):
    chunk, lane = divmod(r, _NLANE)
    base = chunk * 8
    sf = aref[base + 4, lane]
    a5 = aref[base + 5, lane].astype(jnp.int32) * 5
    gj = aref[base + 6, lane].astype(jnp.int32)
    gi = aref[base + 7, lane].astype(jnp.int32)
    return sf, a5, gj, gi


def _mk_copy(r, aref, o0, o1, o2, gs, sem):
    sf, a5, gj, gi = _row_scalars(r, aref)
    si = sf.astype(jnp.int32)
    b8 = (r // 8) * 8
    cps = [
        pltpu.make_async_copy(
            o0.at[pl.ds(a5, 5), gj, pl.ds(b8, 8), :], gs[0].at[r], sem),
        pltpu.make_async_copy(
            o1.at[pl.ds(a5, 5), gj, pl.ds(b8, 8), :], gs[1].at[r], sem),
        pltpu.make_async_copy(
            o2.at[gj, gi, :, :], gs[2].at[r], sem),
    ]
    return [(si == i, cp) for i, cp in enumerate(cps)]


def _comb_body(aref, lse_ref, o0, o1, o2, o_loss, o_conf, o_off,
               g0, g1, g2, sem):
    gs = (g0, g1, g2)
    for r in range(_B):
        for cond, cp in _mk_copy(r, aref, o0, o1, o2, gs, sem):
            pl.when(cond)(cp.start)
    for r in range(_B):
        for cond, cp in _mk_copy(r, aref, o0, o1, o2, gs, sem):
            pl.when(cond)(cp.wait)

    l76 = lax.broadcasted_iota(jnp.int32, (1, 76), 1)
    l38 = lax.broadcasted_iota(jnp.int32, (1, 38), 1)
    ri15 = lax.broadcasted_iota(jnp.int32, (15, _B), 0)
    ci32 = lax.broadcasted_iota(jnp.int32, (15, _B), 1)
    bi = lax.broadcasted_iota(jnp.int32, (1, _B), 1)

    p = [jnp.zeros((1, _B), jnp.float32) for _ in range(5)]
    tx = jnp.zeros((1, _B), jnp.float32)
    ty = jnp.zeros((1, _B), jnp.float32)
    rw = jnp.zeros((1, _B), jnp.float32)
    rh = jnp.zeros((1, _B), jnp.float32)
    for r in range(_B):
        chunk, lane = divmod(r, _NLANE)
        base = chunk * 8
        sf, a5, gj, gi = _row_scalars(r, aref)
        rs = r % 8
        m0 = jnp.where(l76 == gi, 1.0, 0.0)
        m1 = jnp.where(l38 == gi, 1.0, 0.0)
        oh = jnp.where(bi == r, 1.0, 0.0)
        for c in range(5):
            v0 = jnp.sum(g0[r, c, rs:rs + 1, :] * m0)
            v1 = jnp.sum(g1[r, c, rs:rs + 1, :] * m1)
            m2 = jnp.where((ri15 == a5 + c) & (ci32 == r), 1.0, 0.0)
            v2 = jnp.sum(g2[r] * m2)
            val = jnp.where(sf == 0.0, v0, jnp.where(sf == 1.0, v1, v2))
            p[c] = p[c] + val * oh
        tx = tx + aref[base + 0, lane] * oh
        ty = ty + aref[base + 1, lane] * oh
        rw = rw + aref[base + 2, lane] * oh
        rh = rh + aref[base + 3, lane] * oh

    tw = jnp.log(rw + 1e-16)
    th = jnp.log(rh + 1e-16)

    def sig(x):
        return jnp.clip(jax.nn.sigmoid(x), 0.0001, 1.0 - 0.0001)

    off_per = ((sig(p[0]) - tx) ** 2 + (sig(p[1]) - ty) ** 2 +
               (p[2] - tw) ** 2 + (p[3] - th) ** 2)
    off = jnp.sum(off_per, axis=1, keepdims=True) / jnp.float32(_B)
    lc = jnp.sum(lse_ref[...] - p[4], axis=1, keepdims=True) / jnp.float32(_B)
    o_off[...] = off
    o_conf[...] = lc
    o_loss[...] = off + lc


def kernel(out0, out1, out2, targets):
    ttf = targets.T.reshape(-1)
    asn = _sc_assign(ttf)

    o0t = jnp.transpose(out0, (1, 2, 0, 3))
    o1t = jnp.transpose(out1, (1, 2, 0, 3))
    o2t = jnp.transpose(out2, (2, 3, 1, 0))

    lse = pl.pallas_call(
        _lse_body,
        grid=(3,),
        in_specs=[
            pl.BlockSpec((1, 76, _B, 76), lambda i: (4 + 5 * i, 0, 0, 0)),
            pl.BlockSpec((1, 38, _B, 38), lambda i: (4 + 5 * i, 0, 0, 0)),
            pl.BlockSpec((19, 19, 15, _B), lambda i: (0, 0, 0, 0)),
        ],
        out_specs=pl.BlockSpec((1, _B), lambda i: (0, 0)),
        out_shape=jax.ShapeDtypeStruct((1, _B), jnp.float32),
        scratch_shapes=[pltpu.VMEM((1, _B), jnp.float32),
                        pltpu.VMEM((1, _B), jnp.float32)],
    )(o0t, o1t, o2t)

    loss, lc, off = pl.pallas_call(
        _comb_body,
        grid=(1,),
        in_specs=[pl.BlockSpec(memory_space=pltpu.SMEM),
                  pl.BlockSpec((1, _B), lambda i: (0, 0)),
                  pl.BlockSpec(memory_space=pl.ANY),
                  pl.BlockSpec(memory_space=pl.ANY),
                  pl.BlockSpec(memory_space=pl.ANY)],
        out_specs=[pl.BlockSpec((1, 1), lambda i: (0, 0))] * 3,
        out_shape=[jax.ShapeDtypeStruct((1, 1), jnp.float32)] * 3,
        scratch_shapes=[pltpu.VMEM((_B, 5, 8, _GRIDS[0]), jnp.float32),
                        pltpu.VMEM((_B, 5, 8, _GRIDS[1]), jnp.float32),
                        pltpu.VMEM((_B, 15, _B), jnp.float32),
                        pltpu.SemaphoreType.DMA],
    )(asn, lse, o0t, o1t, o2t)

    return (loss.reshape(1), lc.reshape(1), off.reshape(1))


# speedup vs baseline: 9.2051x; 1.0454x over previous
import functools

import numpy as np
import jax
import jax.numpy as jnp
from jax import lax
from jax.experimental import pallas as pl
from jax.experimental.pallas import tpu as pltpu
from jax.experimental.pallas import tpu_sc as plsc

_IMG = 608.0
_GRIDS = (76, 38, 19)
_ANCH = np.array(
    [[10, 13], [16, 30], [33, 23], [30, 61], [62, 45], [59, 119],
     [116, 90], [156, 198], [373, 326]], dtype=np.float32).reshape(3, 3, 2)
_SCALED = [_ANCH[i] / (_IMG / g) for i, g in enumerate(_GRIDS)]
_B = 32
_NLANE = 16
_NCORE = 2


def _sc_body(tt_hbm, asn_hbm, tv, av):
    wid = lax.axis_index("s") * _NCORE + lax.axis_index("c")
    chunk = wid // _NLANE
    j = wid % _NLANE
    c16 = chunk * _NLANE

    pltpu.sync_copy(tt_hbm, tv)

    x1 = tv[pl.ds(0 * _B + c16, 16)] / _IMG
    y1 = tv[pl.ds(1 * _B + c16, 16)] / _IMG
    x2 = tv[pl.ds(2 * _B + c16, 16)] / _IMG
    y2 = tv[pl.ds(3 * _B + c16, 16)] / _IMG

    biou, ba_s, gj_s, gi_s, fx_s, fy_s, rw_s, rh_s = ([] for _ in range(8))
    for i, nG in enumerate(_GRIDS):
        g = jnp.float32(float(nG))
        tx1 = x1 * g
        ty1 = y1 * g
        tx2 = x2 * g
        ty2 = y2 * g
        gx = (tx1 + tx2) / 2.0
        gy = (ty1 + ty2) / 2.0
        gw = tx2 - tx1
        gh = ty2 - ty1
        wh_area = gw * gh

        best_i = None
        best_a = jnp.zeros((16,), jnp.int32)
        for a in range(3):
            w1 = np.float32(_SCALED[i][a, 0])
            h1 = np.float32(_SCALED[i][a, 1])
            ua = np.float32(w1 * h1 + np.float32(1e-16))
            inter = (jnp.minimum(jnp.float32(w1), gw) *
                     jnp.minimum(jnp.float32(h1), gh))
            iou = inter / (jnp.float32(ua) + wh_area - inter)
            if a == 0:
                best_i = iou
            else:
                upd = iou > best_i
                best_a = jnp.where(upd, jnp.int32(a), best_a)
                best_i = jnp.maximum(best_i, iou)
        biou.append(best_i)
        ba_s.append(best_a)

        gi = gx.astype(jnp.int32)
        gj = gy.astype(jnp.int32)
        gi_s.append(gi.astype(jnp.float32))
        gj_s.append(gj.astype(jnp.float32))
        fx_s.append(gx - gi.astype(jnp.float32))
        fy_s.append(gy - gj.astype(jnp.float32))

        w0 = float(_SCALED[i][0, 0]); h0 = float(_SCALED[i][0, 1])
        w1f = float(_SCALED[i][1, 0]); h1f = float(_SCALED[i][1, 1])
        w2f = float(_SCALED[i][2, 0]); h2f = float(_SCALED[i][2, 1])
        aw = jnp.where(best_a == 0, jnp.float32(w0),
                       jnp.where(best_a == 1, jnp.float32(w1f),
                                 jnp.float32(w2f)))
        ah = jnp.where(best_a == 0, jnp.float32(h0),
                       jnp.where(best_a == 1, jnp.float32(h1f),
                                 jnp.float32(h2f)))
        rw_s.append(gw / aw)
        rh_s.append(gh / ah)

    ssel = jnp.zeros((16,), jnp.int32)
    sbest = biou[0]
    for i in (1, 2):
        upd = biou[i] > sbest
        ssel = jnp.where(upd, jnp.int32(i), ssel)
        sbest = jnp.maximum(sbest, biou[i])

    def sel3(vs):
        return jnp.where(ssel == 0, vs[0],
                         jnp.where(ssel == 1, vs[1], vs[2]))

    @pl.when(j == 0)
    def _():
        av[0, :] = sel3(fx_s)
        av[1, :] = sel3(fy_s)
        av[2, :] = sel3(rw_s)
        av[3, :] = sel3(rh_s)
        av[4, :] = ssel.astype(jnp.float32)
        av[5, :] = sel3(ba_s).astype(jnp.float32)
        av[6, :] = sel3(gj_s)
        av[7, :] = sel3(gi_s)
        pltpu.sync_copy(av, asn_hbm.at[pl.ds(chunk * 8, 8)])


_sc_assign = functools.partial(
    pl.kernel,
    out_type=jax.ShapeDtypeStruct((16, 16), jnp.float32),
    mesh=plsc.VectorSubcoreMesh(core_axis_name="c", subcore_axis_name="s"),
    scratch_types=[pltpu.VMEM((4 * _B,), jnp.float32),
                   pltpu.VMEM((8, 16), jnp.float32)],
    compiler_params=pltpu.CompilerParams(needs_layout_passes=False),
)(_sc_body)


def _lse_body(c0, c1, c2, o_lse, m_sc, se_sc):
    first = pl.program_id(0) == 0
    x0 = c0[0]
    x1 = c1[0]

    def rowmax(x, axes):
        return jnp.max(x, axis=axes).reshape(1, _B)

    mx = jnp.maximum(rowmax(x0, (0, 2)), rowmax(x1, (0, 2)))
    m_old = jnp.where(first, jnp.float32(-1e30), m_sc[...])
    se_old = jnp.where(first, jnp.float32(0.0), se_sc[...])
    m_new = jnp.maximum(m_old, mx)
    se = se_old * jnp.exp(m_old - m_new)
    mv = m_new.reshape(_B)
    se = se + jnp.sum(jnp.exp(x0 - mv[None, :, None]),
                      axis=(0, 2)).reshape(1, _B)
    se = se + jnp.sum(jnp.exp(x1 - mv[None, :, None]),
                      axis=(0, 2)).reshape(1, _B)
    m_sc[...] = m_new
    se_sc[...] = se

    @pl.when(pl.program_id(0) == 2)
    def _():
        X2 = c2[...]
        x2s = [X2[:, :, 4, :], X2[:, :, 9, :], X2[:, :, 14, :]]
        mx2 = jnp.maximum(jnp.maximum(rowmax(x2s[0], (0, 1)),
                                      rowmax(x2s[1], (0, 1))),
                          rowmax(x2s[2], (0, 1)))
        m_f = jnp.maximum(m_new, mx2)
        mvf = m_f.reshape(_B)
        se_f = se * jnp.exp(m_new - m_f)
        for xs in x2s:
            se_f = se_f + jnp.sum(jnp.exp(xs - mvf[None, None, :]),
                                  axis=(0, 1)).reshape(1, _B)
        o_lse[...] = jnp.log(se_f) + m_f


def _row_scalars(r, aref):
    chunk, lane = divmod(r, _NLANE)
    base = chunk * 8
    sf = aref[base + 4, lane]
    a5 = aref[base + 5, lane].astype(jnp.int32) * 5
    gj = aref[base + 6, lane].astype(jnp.int32)
    gi = aref[base + 7, lane].astype(jnp.int32)
    return sf, a5, gj, gi


def _mk_copy(r, aref, o0, o1, o2, gs, sem):
    sf, a5, gj, gi = _row_scalars(r, aref)
    si = sf.astype(jnp.int32)
    b8 = (r // 8) * 8
    cps = [
        pltpu.make_async_copy(
            o0.at[pl.ds(a5, 5), gj, pl.ds(b8, 8), :], gs[0].at[r], sem),
        pltpu.make_async_copy(
            o1.at[pl.ds(a5, 5), gj, pl.ds(b8, 8), :], gs[1].at[r], sem),
        pltpu.make_async_copy(
            o2.at[gj, gi, :, :], gs[2].at[r], sem),
    ]
    return [(si == i, cp) for i, cp in enumerate(cps)]


def _comb_body(aref, lse_ref, o0, o1, o2, o_loss, o_conf, o_off,
               g0, g1, g2, sem):
    gs = (g0, g1, g2)
    descs = [_mk_copy(r, aref, o0, o1, o2, gs, sem) for r in range(_B)]
    for row in descs:
        for cond, cp in row:
            pl.when(cond)(cp.start)
    for row in descs:
        for cond, cp in row:
            pl.when(cond)(cp.wait)

    l76 = lax.broadcasted_iota(jnp.int32, (1, 76), 1)
    l38 = lax.broadcasted_iota(jnp.int32, (1, 38), 1)
    ri15 = lax.broadcasted_iota(jnp.int32, (15, _B), 0)
    ci32 = lax.broadcasted_iota(jnp.int32, (15, _B), 1)
    bi = lax.broadcasted_iota(jnp.int32, (1, _B), 1)

    p = [jnp.zeros((1, _B), jnp.float32) for _ in range(5)]
    tx = jnp.zeros((1, _B), jnp.float32)
    ty = jnp.zeros((1, _B), jnp.float32)
    rw = jnp.zeros((1, _B), jnp.float32)
    rh = jnp.zeros((1, _B), jnp.float32)
    for r in range(_B):
        chunk, lane = divmod(r, _NLANE)
        base = chunk * 8
        sf, a5, gj, gi = _row_scalars(r, aref)
        rs = r % 8
        m0 = jnp.where(l76 == gi, 1.0, 0.0)
        m1 = jnp.where(l38 == gi, 1.0, 0.0)
        oh = jnp.where(bi == r, 1.0, 0.0)
        for c in range(5):
            v0 = jnp.sum(g0[r, c, rs:rs + 1, :] * m0)
            v1 = jnp.sum(g1[r, c, rs:rs + 1, :] * m1)
            m2 = jnp.where((ri15 == a5 + c) & (ci32 == r), 1.0, 0.0)
            v2 = jnp.sum(g2[r] * m2)
            val = jnp.where(sf == 0.0, v0, jnp.where(sf == 1.0, v1, v2))
            p[c] = p[c] + val * oh
        tx = tx + aref[base + 0, lane] * oh
        ty = ty + aref[base + 1, lane] * oh
        rw = rw + aref[base + 2, lane] * oh
        rh = rh + aref[base + 3, lane] * oh

    tw = jnp.log(rw + 1e-16)
    th = jnp.log(rh + 1e-16)

    def sig(x):
        return jnp.clip(jax.nn.sigmoid(x), 0.0001, 1.0 - 0.0001)

    off_per = ((sig(p[0]) - tx) ** 2 + (sig(p[1]) - ty) ** 2 +
               (p[2] - tw) ** 2 + (p[3] - th) ** 2)
    off = jnp.sum(off_per, axis=1, keepdims=True) / jnp.float32(_B)
    lc = jnp.sum(lse_ref[...] - p[4], axis=1, keepdims=True) / jnp.float32(_B)
    o_off[...] = off
    o_conf[...] = lc
    o_loss[...] = off + lc


def kernel(out0, out1, out2, targets):
    ttf = targets.T.reshape(-1)
    asn = _sc_assign(ttf)

    o0t = jnp.transpose(out0, (1, 2, 0, 3))
    o1t = jnp.transpose(out1, (1, 2, 0, 3))
    o2t = jnp.transpose(out2, (2, 3, 1, 0))

    lse = pl.pallas_call(
        _lse_body,
        grid=(3,),
        in_specs=[
            pl.BlockSpec((1, 76, _B, 76), lambda i: (4 + 5 * i, 0, 0, 0)),
            pl.BlockSpec((1, 38, _B, 38), lambda i: (4 + 5 * i, 0, 0, 0)),
            pl.BlockSpec((19, 19, 15, _B), lambda i: (0, 0, 0, 0)),
        ],
        out_specs=pl.BlockSpec((1, _B), lambda i: (0, 0)),
        out_shape=jax.ShapeDtypeStruct((1, _B), jnp.float32),
        scratch_shapes=[pltpu.VMEM((1, _B), jnp.float32),
                        pltpu.VMEM((1, _B), jnp.float32)],
    )(o0t, o1t, o2t)

    loss, lc, off = pl.pallas_call(
        _comb_body,
        grid=(1,),
        in_specs=[pl.BlockSpec(memory_space=pltpu.SMEM),
                  pl.BlockSpec((1, _B), lambda i: (0, 0)),
                  pl.BlockSpec(memory_space=pl.ANY),
                  pl.BlockSpec(memory_space=pl.ANY),
                  pl.BlockSpec(memory_space=pl.ANY)],
        out_specs=[pl.BlockSpec((1, 1), lambda i: (0, 0))] * 3,
        out_shape=[jax.ShapeDtypeStruct((1, 1), jnp.float32)] * 3,
        scratch_shapes=[pltpu.VMEM((_B, 5, 8, _GRIDS[0]), jnp.float32),
                        pltpu.VMEM((_B, 5, 8, _GRIDS[1]), jnp.float32),
                        pltpu.VMEM((_B, 15, _B), jnp.float32),
                        pltpu.SemaphoreType.DMA],
    )(asn, lse, o0t, o1t, o2t)

    return (loss.reshape(1), lc.reshape(1), off.reshape(1))
